# Initial kernel scaffold; baseline (speedup 1.0000x reference)
import jax
import jax.numpy as jnp
from jax.experimental import pallas as pl
from jax.experimental.pallas import tpu as pltpu
from jax.experimental.pallas import tpu_sc as plsc

_WINDOW = 128


def kernel(inputs, lookup_table):
    batch, seq = inputs.shape
    vocab, dim = lookup_table.shape
    n = batch * seq
    assert n % _WINDOW == 0

    mesh = plsc.VectorSubcoreMesh(core_axis_name="c", subcore_axis_name="s")
    idx = inputs.reshape(1, n).astype(jnp.int32)

    @pl.kernel(
        out_type=jax.ShapeDtypeStruct((n, dim), lookup_table.dtype),
        mesh=mesh,
    )
    def gather_kernel(table_hbm, i_hbm, o_hbm):
        def body(i_vmem, o_vmem):
            pltpu.sync_copy(table_hbm.at[i_vmem.at[0]], o_vmem)

        pltpu.emit_pipeline(
            body,
            grid=(n // _WINDOW,),
            in_specs=[pl.BlockSpec((1, _WINDOW), lambda i: (0, i))],
            out_specs=[pl.BlockSpec((_WINDOW, dim), lambda i: (i, 0))],
            core_axis_name=("c", "s"),
            dimension_semantics=(pltpu.PARALLEL,),
        )(i_hbm, o_hbm)

    out = gather_kernel(lookup_table, idx)
    return out.reshape(batch, seq, dim)


# speedup vs baseline: 1.2512x; 1.2512x over previous
import jax
import jax.numpy as jnp
from jax import lax
from jax.experimental import pallas as pl
from jax.experimental.pallas import tpu as pltpu
from jax.experimental.pallas import tpu_sc as plsc

_NUM_CORES = 2
_NUM_SUBCORES = 16
_NUM_WORKERS = _NUM_CORES * _NUM_SUBCORES
_CHUNK = 128


def kernel(inputs, lookup_table):
    batch, seq = inputs.shape
    vocab, dim = lookup_table.shape
    n = batch * seq
    per_worker = n // _NUM_WORKERS
    assert n % (_NUM_WORKERS * _CHUNK) == 0

    mesh = plsc.VectorSubcoreMesh(core_axis_name="c", subcore_axis_name="s")
    idx = inputs.reshape(n).astype(jnp.int32)
    table128 = jnp.pad(lookup_table, ((0, 0), (0, 128 - dim)))

    @pl.kernel(
        out_type=jax.ShapeDtypeStruct((n, 128), lookup_table.dtype),
        mesh=mesh,
        scratch_types=[
            pltpu.VMEM((_CHUNK,), jnp.int32),
            pltpu.VMEM((_CHUNK, 128), jnp.float32),
            pltpu.SemaphoreType.DMA,
        ],
    )
    def gather_kernel(table_hbm, idx_hbm, out_hbm, idx_v, rows_v, sem):
        wid = lax.axis_index("s") * _NUM_CORES + lax.axis_index("c")
        base = wid * per_worker

        @pl.loop(0, per_worker, step=_CHUNK)
        def _(off):
            b = base + off
            pltpu.sync_copy(idx_hbm.at[pl.ds(b, _CHUNK)], idx_v)
            pltpu.async_copy(table_hbm.at[idx_v], rows_v, sem).wait()
            pltpu.sync_copy(rows_v, out_hbm.at[pl.ds(b, _CHUNK)])

    out = gather_kernel(table128, idx)
    return out[:, :dim].reshape(batch, seq, dim)
